# baseline (speedup 1.0000x reference)
import jax
import jax.numpy as jnp
from jax import lax
from jax.experimental import pallas as pl
from jax.experimental.pallas import tpu as pltpu

_N = 16777216
_C = 8192
_R = _N // _C
_BM = 64
_GRID = _R // _BM


def _fill_body(idx_ref, grad_ref, out_ref):
    pid = pl.program_id(0)
    out_ref[...] = jnp.zeros_like(out_ref)
    target = idx_ref[0]
    row = target // _C
    col = target % _C
    row0 = pid * _BM

    @pl.when((row >= row0) & (row < row0 + _BM))
    def _():
        r = row - row0
        cols = lax.broadcasted_iota(jnp.int32, (1, _C), 1)
        out_ref[pl.ds(r, 1), :] = jnp.where(cols == col, grad_ref[0], 0.0)


def kernel(grad_output, input_sizes, dim, index, out):
    n = out.shape[0]
    idx = ((jnp.asarray(index, jnp.int32) + jnp.asarray(dim, jnp.int32))
           % jnp.asarray(input_sizes, jnp.int32)).reshape((1,))
    gval = jnp.asarray(grad_output, jnp.float32).reshape((1,))
    res = pl.pallas_call(
        _fill_body,
        grid=(_GRID,),
        in_specs=[pl.BlockSpec(memory_space=pltpu.SMEM),
                  pl.BlockSpec(memory_space=pltpu.SMEM)],
        out_specs=pl.BlockSpec((_BM, _C), lambda i: (i, 0)),
        out_shape=jax.ShapeDtypeStruct((_R, _C), jnp.float32),
    )(idx, gval)
    return res.reshape(n)


# speedup vs baseline: 1.0212x; 1.0212x over previous
import jax
import jax.numpy as jnp
from jax import lax
from jax.experimental import pallas as pl
from jax.experimental.pallas import tpu as pltpu

_N = 16777216
_C = 8192
_R = _N // _C
_BM = 64
_NCOPIES = _R // _BM


def _fill_body(idx_ref, grad_ref, out_ref, zbuf, gbuf, sem):
    target = idx_ref[0]
    row = target // _C
    col = target % _C
    kstar = row // _BM
    r_loc = row % _BM

    zbuf[...] = jnp.zeros_like(zbuf)
    rows_i = lax.broadcasted_iota(jnp.int32, (_BM, _C), 0)
    cols_i = lax.broadcasted_iota(jnp.int32, (_BM, _C), 1)
    gbuf[...] = jnp.where((rows_i == r_loc) & (cols_i == col), grad_ref[0], 0.0)

    copies = []
    for k in range(_NCOPIES):
        dst = out_ref.at[pl.ds(k * _BM, _BM)]
        zc = pltpu.make_async_copy(zbuf, dst, sem)
        gc = pltpu.make_async_copy(gbuf, dst, sem)

        @pl.when(kstar != k)
        def _():
            zc.start()

        @pl.when(kstar == k)
        def _():
            gc.start()

        copies.append(zc)
    for c in copies:
        c.wait()


def kernel(grad_output, input_sizes, dim, index, out):
    n = out.shape[0]
    idx = ((jnp.asarray(index, jnp.int32) + jnp.asarray(dim, jnp.int32))
           % jnp.asarray(input_sizes, jnp.int32)).reshape((1,))
    gval = jnp.asarray(grad_output, jnp.float32).reshape((1,))
    res = pl.pallas_call(
        _fill_body,
        in_specs=[pl.BlockSpec(memory_space=pltpu.SMEM),
                  pl.BlockSpec(memory_space=pltpu.SMEM)],
        out_specs=pl.BlockSpec(memory_space=pl.ANY),
        out_shape=jax.ShapeDtypeStruct((_R, _C), jnp.float32),
        scratch_shapes=[
            pltpu.VMEM((_BM, _C), jnp.float32),
            pltpu.VMEM((_BM, _C), jnp.float32),
            pltpu.SemaphoreType.DMA,
        ],
    )(idx, gval)
    return res.reshape(n)


# speedup vs baseline: 3.4632x; 3.3913x over previous
import jax
import jax.numpy as jnp
from jax import lax
from jax.experimental import pallas as pl
from jax.experimental.pallas import tpu as pltpu

_N = 16777216
_CH = 524288
_NCOPIES = _N // _CH


def _fill_body(idx_ref, grad_ref, out_ref, zbuf, gbuf, sem):
    target = idx_ref[0]
    kstar = target // _CH
    off = target % _CH

    zbuf[...] = jnp.zeros_like(zbuf)
    pos = lax.broadcasted_iota(jnp.int32, (_CH,), 0)
    gbuf[...] = jnp.where(pos == off, grad_ref[0], 0.0)

    copies = []
    for k in range(_NCOPIES):
        dst = out_ref.at[pl.ds(k * _CH, _CH)]
        zc = pltpu.make_async_copy(zbuf, dst, sem)
        gc = pltpu.make_async_copy(gbuf, dst, sem)

        @pl.when(kstar != k)
        def _():
            zc.start()

        @pl.when(kstar == k)
        def _():
            gc.start()

        copies.append(zc)
    for c in copies:
        c.wait()


def kernel(grad_output, input_sizes, dim, index, out):
    n = out.shape[0]
    idx = ((jnp.asarray(index, jnp.int32) + jnp.asarray(dim, jnp.int32))
           % jnp.asarray(input_sizes, jnp.int32)).reshape((1,))
    gval = jnp.asarray(grad_output, jnp.float32).reshape((1,))
    res = pl.pallas_call(
        _fill_body,
        in_specs=[pl.BlockSpec(memory_space=pltpu.SMEM),
                  pl.BlockSpec(memory_space=pltpu.SMEM)],
        out_specs=pl.BlockSpec(memory_space=pl.ANY),
        out_shape=jax.ShapeDtypeStruct((n,), jnp.float32),
        scratch_shapes=[
            pltpu.VMEM((_CH,), jnp.float32),
            pltpu.VMEM((_CH,), jnp.float32),
            pltpu.SemaphoreType.DMA,
        ],
    )(idx, gval)
    return res
